# baseline (speedup 1.0000x reference)
import functools

import jax
import jax.numpy as jnp
from jax import lax
from jax.experimental import pallas as pl
from jax.experimental.pallas import tpu as pltpu
from jax.experimental.pallas import tpu_sc as plsc

_NC = 2
_NS = 16
_K = 128
_C = 16


def _sc_aggregate(x, src, dst, zeros, *, n_pad, rps, bpw):
    d = x.shape[1]
    mesh = plsc.VectorSubcoreMesh(core_axis_name="c", subcore_axis_name="s")

    @functools.partial(
        pl.kernel,
        out_type=jax.ShapeDtypeStruct((_NC * n_pad, d), jnp.float32),
        mesh=mesh,
        scratch_types=[
            pltpu.VMEM((_C, _K), jnp.int32),
            pltpu.VMEM((_C, _K), jnp.int32),
            pltpu.VMEM((_K, d), jnp.float32),
            pltpu.VMEM((_K, d), jnp.float32),
            pltpu.VMEM_SHARED((n_pad, d), jnp.float32),
            pltpu.SemaphoreType.DMA,
            pltpu.SemaphoreType.DMA,
        ],
    )
    def agg_kernel(x_hbm, src_hbm, dst_hbm, z_hbm, out_hbm,
                   sidx, didx, rows_a, rows_b, acc, sem_a, sem_b):
        cid = lax.axis_index("c")
        sid = lax.axis_index("s")
        wid = sid * _NC + cid
        base = wid * bpw

        pltpu.sync_copy(z_hbm, acc.at[pl.ds(sid * rps, rps)])
        plsc.subcore_barrier()

        nb2 = _C // 2

        @pl.loop(0, bpw // _C)
        def _(ch):
            blk0 = base + ch * _C
            pltpu.sync_copy(src_hbm.at[pl.ds(blk0, _C)], sidx)
            pltpu.sync_copy(dst_hbm.at[pl.ds(blk0, _C)], didx)

            @pl.loop(0, _C)
            def _(j):
                pltpu.sync_copy(x_hbm.at[sidx.at[j]], rows_a)
                pltpu.sync_copy(rows_a, acc.at[didx.at[j]], add=True)

        plsc.subcore_barrier()
        pltpu.sync_copy(
            acc.at[pl.ds(sid * rps, rps)],
            out_hbm.at[pl.ds(cid * n_pad + sid * rps, rps)],
        )

    return agg_kernel(x, src, dst, zeros)


def _mlp(x, p0, p1, W1, b1, W2, b2):
    n, d = x.shape
    r = 1000
    assert n % r == 0

    def body(x_ref, p0_ref, p1_ref, w1_ref, b1_ref, w2_ref, b2_ref, o_ref):
        h = x_ref[...] + p0_ref[...] + p1_ref[...]
        h = jnp.dot(h, w1_ref[...], preferred_element_type=jnp.float32)
        h = jnp.maximum(h + b1_ref[...], 0.0)
        o = jnp.dot(h, w2_ref[...], preferred_element_type=jnp.float32)
        o_ref[...] = o + b2_ref[...]

    return pl.pallas_call(
        body,
        grid=(n // r,),
        in_specs=[
            pl.BlockSpec((r, d), lambda i: (i, 0)),
            pl.BlockSpec((r, d), lambda i: (i, 0)),
            pl.BlockSpec((r, d), lambda i: (i, 0)),
            pl.BlockSpec((d, d), lambda i: (0, 0)),
            pl.BlockSpec((1, d), lambda i: (0, 0)),
            pl.BlockSpec((d, d), lambda i: (0, 0)),
            pl.BlockSpec((1, d), lambda i: (0, 0)),
        ],
        out_specs=pl.BlockSpec((r, d), lambda i: (i, 0)),
        out_shape=jax.ShapeDtypeStruct((n, d), jnp.float32),
    )(x, p0, p1, W1, b1.reshape(1, d), W2, b2.reshape(1, d))


def kernel(x, edge_index, W1, b1, W2, b2):
    n, d = x.shape
    e = edge_index.shape[1]

    rps = -(-(n + 1) // _NS)
    rps = -(-rps // 8) * 8
    n_pad = _NS * rps

    eb = _NC * _NS * _K * _C
    e_pad = -(-e // eb) * eb
    src = edge_index[0]
    dst = edge_index[1]
    if e_pad != e:
        pad = e_pad - e
        src = jnp.concatenate([src, jnp.zeros((pad,), jnp.int32)])
        dst = jnp.concatenate([dst, jnp.full((pad,), n, jnp.int32)])
    bpw = e_pad // (_NC * _NS * _K)
    src = src.reshape(e_pad // _K, _K)
    dst = dst.reshape(e_pad // _K, _K)

    zeros = jnp.zeros((rps, d), jnp.float32)
    partials = _sc_aggregate(x, src, dst, zeros, n_pad=n_pad, rps=rps, bpw=bpw)
    p0 = partials[:n]
    p1 = partials[n_pad:n_pad + n]
    return _mlp(x, p0, p1, W1, b1, W2, b2)


# speedup vs baseline: 1.0158x; 1.0158x over previous
import functools

import jax
import jax.numpy as jnp
from jax import lax
from jax.experimental import pallas as pl
from jax.experimental.pallas import tpu as pltpu
from jax.experimental.pallas import tpu_sc as plsc

_NC = 2
_NS = 16
_K = 128
_C = 16


def _sc_aggregate(x, sd, zeros, *, n_pad, rps, bpw):
    d = x.shape[1]
    mesh = plsc.VectorSubcoreMesh(core_axis_name="c", subcore_axis_name="s")

    @functools.partial(
        pl.kernel,
        out_type=jax.ShapeDtypeStruct((_NC * n_pad, d), jnp.float32),
        mesh=mesh,
        scratch_types=[
            pltpu.VMEM((2, _K), jnp.int32),
            pltpu.VMEM((2, _K), jnp.int32),
            pltpu.VMEM((_K, d), jnp.float32),
            pltpu.VMEM((_K, d), jnp.float32),
            pltpu.VMEM_SHARED((n_pad, d), jnp.float32),
            pltpu.SemaphoreType.DMA,
            pltpu.SemaphoreType.DMA,
            pltpu.SemaphoreType.DMA,
            pltpu.SemaphoreType.DMA,
        ],
    )
    def agg_kernel(x_hbm, sd_hbm, z_hbm, out_hbm,
                   idx_a, idx_b, rows_a, rows_b, acc,
                   isem_a, isem_b, gsem_a, gsem_b):
        cid = lax.axis_index("c")
        sid = lax.axis_index("s")
        wid = sid * _NC + cid
        base = wid * bpw

        pltpu.sync_copy(z_hbm, acc.at[pl.ds(sid * rps, rps)])
        plsc.subcore_barrier()

        nb2 = bpw // 2

        @pl.loop(0, nb2)
        def _(j):
            b0 = base + 2 * j
            b1 = b0 + 1
            ia = pltpu.async_copy(sd_hbm.at[pl.ds(2 * b0, 2)], idx_a, isem_a)
            ib = pltpu.async_copy(sd_hbm.at[pl.ds(2 * b1, 2)], idx_b, isem_b)
            ia.wait()
            ga = pltpu.async_copy(x_hbm.at[idx_a.at[0]], rows_a, gsem_a)
            ib.wait()
            gb = pltpu.async_copy(x_hbm.at[idx_b.at[0]], rows_b, gsem_b)
            ga.wait()
            pltpu.sync_copy(rows_a, acc.at[idx_a.at[1]], add=True)
            gb.wait()
            pltpu.sync_copy(rows_b, acc.at[idx_b.at[1]], add=True)

        plsc.subcore_barrier()
        pltpu.sync_copy(
            acc.at[pl.ds(sid * rps, rps)],
            out_hbm.at[pl.ds(cid * n_pad + sid * rps, rps)],
        )

    return agg_kernel(x, sd, zeros)


def _mlp(x, p0, p1, W1, b1, W2, b2):
    n, d = x.shape
    r = 1000
    assert n % r == 0

    def body(x_ref, p0_ref, p1_ref, w1_ref, b1_ref, w2_ref, b2_ref, o_ref):
        h = x_ref[...] + p0_ref[...] + p1_ref[...]
        h = jnp.dot(h, w1_ref[...], preferred_element_type=jnp.float32)
        h = jnp.maximum(h + b1_ref[...], 0.0)
        o = jnp.dot(h, w2_ref[...], preferred_element_type=jnp.float32)
        o_ref[...] = o + b2_ref[...]

    return pl.pallas_call(
        body,
        grid=(n // r,),
        in_specs=[
            pl.BlockSpec((r, d), lambda i: (i, 0)),
            pl.BlockSpec((r, d), lambda i: (i, 0)),
            pl.BlockSpec((r, d), lambda i: (i, 0)),
            pl.BlockSpec((d, d), lambda i: (0, 0)),
            pl.BlockSpec((1, d), lambda i: (0, 0)),
            pl.BlockSpec((d, d), lambda i: (0, 0)),
            pl.BlockSpec((1, d), lambda i: (0, 0)),
        ],
        out_specs=pl.BlockSpec((r, d), lambda i: (i, 0)),
        out_shape=jax.ShapeDtypeStruct((n, d), jnp.float32),
    )(x, p0, p1, W1, b1.reshape(1, d), W2, b2.reshape(1, d))


def kernel(x, edge_index, W1, b1, W2, b2):
    n, d = x.shape
    e = edge_index.shape[1]

    rps = -(-(n + 1) // _NS)
    rps = -(-rps // 8) * 8
    n_pad = _NS * rps

    eb = _NC * _NS * _K * 2
    e_pad = -(-e // eb) * eb
    src = edge_index[0]
    dst = edge_index[1]
    if e_pad != e:
        pad = e_pad - e
        src = jnp.concatenate([src, jnp.zeros((pad,), jnp.int32)])
        dst = jnp.concatenate([dst, jnp.full((pad,), n, jnp.int32)])
    bpw = e_pad // (_NC * _NS * _K)
    nb = e_pad // _K
    sd = jnp.stack([src.reshape(nb, _K), dst.reshape(nb, _K)], axis=1)
    sd = sd.reshape(2 * nb, _K)

    zeros = jnp.zeros((rps, d), jnp.float32)
    partials = _sc_aggregate(x, sd, zeros, n_pad=n_pad, rps=rps, bpw=bpw)
    p0 = partials[:n]
    p1 = partials[n_pad:n_pad + n]
    return _mlp(x, p0, p1, W1, b1, W2, b2)


# speedup vs baseline: 1.0187x; 1.0029x over previous
import functools

import jax
import jax.numpy as jnp
from jax import lax
from jax.experimental import pallas as pl
from jax.experimental.pallas import tpu as pltpu
from jax.experimental.pallas import tpu_sc as plsc

_NC = 2
_NS = 16
_K = 128
_C = 16


def _sc_aggregate(x, src, dst, zeros, *, n_pad, rps, bpw):
    d = x.shape[1]
    mesh = plsc.VectorSubcoreMesh(core_axis_name="c", subcore_axis_name="s")

    @functools.partial(
        pl.kernel,
        out_type=jax.ShapeDtypeStruct((_NC * n_pad, d), jnp.float32),
        mesh=mesh,
        scratch_types=[
            pltpu.VMEM((_K,), jnp.int32),
            pltpu.VMEM((_K,), jnp.int32),
            pltpu.VMEM((_K,), jnp.int32),
            pltpu.VMEM((_K,), jnp.int32),
            pltpu.VMEM((_K, d), jnp.float32),
            pltpu.VMEM((_K, d), jnp.float32),
            pltpu.VMEM_SHARED((n_pad, d), jnp.float32),
            pltpu.SemaphoreType.DMA,
            pltpu.SemaphoreType.DMA,
        ],
    )
    def agg_kernel(x_hbm, src_hbm, dst_hbm, z_hbm, out_hbm,
                   sidx_a, didx_a, sidx_b, didx_b, rows_a, rows_b, acc,
                   gsem_a, gsem_b):
        cid = lax.axis_index("c")
        sid = lax.axis_index("s")
        wid = sid * _NC + cid
        base = wid * bpw

        pltpu.sync_copy(z_hbm, acc.at[pl.ds(sid * rps, rps)])
        plsc.subcore_barrier()

        nb2 = bpw // 2

        @pl.loop(0, nb2)
        def _(j):
            o0 = (base + 2 * j) * _K
            o1 = o0 + _K
            pltpu.sync_copy(src_hbm.at[pl.ds(o0, _K)], sidx_a)
            pltpu.sync_copy(dst_hbm.at[pl.ds(o0, _K)], didx_a)
            ga = pltpu.async_copy(x_hbm.at[sidx_a], rows_a, gsem_a)
            pltpu.sync_copy(src_hbm.at[pl.ds(o1, _K)], sidx_b)
            pltpu.sync_copy(dst_hbm.at[pl.ds(o1, _K)], didx_b)
            gb = pltpu.async_copy(x_hbm.at[sidx_b], rows_b, gsem_b)
            ga.wait()
            pltpu.sync_copy(rows_a, acc.at[didx_a], add=True)
            gb.wait()
            pltpu.sync_copy(rows_b, acc.at[didx_b], add=True)

        plsc.subcore_barrier()
        pltpu.sync_copy(
            acc.at[pl.ds(sid * rps, rps)],
            out_hbm.at[pl.ds(cid * n_pad + sid * rps, rps)],
        )

    return agg_kernel(x, src, dst, zeros)


def _mlp(x, p0, p1, W1, b1, W2, b2):
    n, d = x.shape
    r = 1000
    assert n % r == 0

    def body(x_ref, p0_ref, p1_ref, w1_ref, b1_ref, w2_ref, b2_ref, o_ref):
        h = x_ref[...] + p0_ref[...] + p1_ref[...]
        h = jnp.dot(h, w1_ref[...], preferred_element_type=jnp.float32)
        h = jnp.maximum(h + b1_ref[...], 0.0)
        o = jnp.dot(h, w2_ref[...], preferred_element_type=jnp.float32)
        o_ref[...] = o + b2_ref[...]

    return pl.pallas_call(
        body,
        grid=(n // r,),
        in_specs=[
            pl.BlockSpec((r, d), lambda i: (i, 0)),
            pl.BlockSpec((r, d), lambda i: (i, 0)),
            pl.BlockSpec((r, d), lambda i: (i, 0)),
            pl.BlockSpec((d, d), lambda i: (0, 0)),
            pl.BlockSpec((1, d), lambda i: (0, 0)),
            pl.BlockSpec((d, d), lambda i: (0, 0)),
            pl.BlockSpec((1, d), lambda i: (0, 0)),
        ],
        out_specs=pl.BlockSpec((r, d), lambda i: (i, 0)),
        out_shape=jax.ShapeDtypeStruct((n, d), jnp.float32),
    )(x, p0, p1, W1, b1.reshape(1, d), W2, b2.reshape(1, d))


def kernel(x, edge_index, W1, b1, W2, b2):
    n, d = x.shape
    e = edge_index.shape[1]

    rps = -(-(n + 1) // _NS)
    rps = -(-rps // 8) * 8
    n_pad = _NS * rps

    eb = _NC * _NS * _K * 2
    e_pad = -(-e // eb) * eb
    src = edge_index[0]
    dst = edge_index[1]
    if e_pad != e:
        pad = e_pad - e
        src = jnp.concatenate([src, jnp.zeros((pad,), jnp.int32)])
        dst = jnp.concatenate([dst, jnp.full((pad,), n, jnp.int32)])
    bpw = e_pad // (_NC * _NS * _K)

    zeros = jnp.zeros((rps, d), jnp.float32)
    partials = _sc_aggregate(x, src, dst, zeros, n_pad=n_pad, rps=rps, bpw=bpw)
    p0 = partials[:n]
    p1 = partials[n_pad:n_pad + n]
    return _mlp(x, p0, p1, W1, b1, W2, b2)


# speedup vs baseline: 1.5517x; 1.5232x over previous
import functools

import jax
import jax.numpy as jnp
from jax import lax
from jax.experimental import pallas as pl
from jax.experimental.pallas import tpu as pltpu
from jax.experimental.pallas import tpu_sc as plsc

_NC = 2
_NS = 16
_K = 128


def _sc_aggregate(x, src, dst, zeros, *, n_pad, rps, bpw):
    d = x.shape[1]
    mesh = plsc.VectorSubcoreMesh(core_axis_name="c", subcore_axis_name="s")

    @functools.partial(
        pl.kernel,
        out_type=jax.ShapeDtypeStruct((_NC * n_pad, d), jnp.float32),
        mesh=mesh,
        scratch_types=[
            pltpu.VMEM((bpw, 1, _K), jnp.int32),
            pltpu.VMEM((bpw, 1, _K), jnp.int32),
            pltpu.VMEM((_K, d), jnp.float32),
            pltpu.VMEM_SHARED((n_pad, d), jnp.float32),
        ],
    )
    def agg_kernel(x_hbm, src_hbm, dst_hbm, z_hbm, out_hbm,
                   sidx, didx, rows, acc):
        cid = lax.axis_index("c")
        sid = lax.axis_index("s")
        wid = sid * _NC + cid
        base = wid * bpw

        pltpu.sync_copy(z_hbm, acc.at[pl.ds(sid * rps, rps)])
        pltpu.sync_copy(src_hbm.at[pl.ds(base, bpw)], sidx)
        pltpu.sync_copy(dst_hbm.at[pl.ds(base, bpw)], didx)
        plsc.subcore_barrier()

        for j in range(bpw):
            pltpu.sync_copy(x_hbm.at[sidx.at[j, 0]], rows)
            pltpu.sync_copy(rows, acc.at[didx.at[j, 0]], add=True)

        plsc.subcore_barrier()
        pltpu.sync_copy(
            acc.at[pl.ds(sid * rps, rps)],
            out_hbm.at[pl.ds(cid * n_pad + sid * rps, rps)],
        )

    return agg_kernel(x, src, dst, zeros)


def _mlp(x, p0, p1, W1, b1, W2, b2):
    n, d = x.shape
    r = 1000
    assert n % r == 0

    def body(x_ref, p0_ref, p1_ref, w1_ref, b1_ref, w2_ref, b2_ref, o_ref):
        h = x_ref[...] + p0_ref[...] + p1_ref[...]
        h = jnp.dot(h, w1_ref[...], preferred_element_type=jnp.float32)
        h = jnp.maximum(h + b1_ref[...], 0.0)
        o = jnp.dot(h, w2_ref[...], preferred_element_type=jnp.float32)
        o_ref[...] = o + b2_ref[...]

    return pl.pallas_call(
        body,
        grid=(n // r,),
        in_specs=[
            pl.BlockSpec((r, d), lambda i: (i, 0)),
            pl.BlockSpec((r, d), lambda i: (i, 0)),
            pl.BlockSpec((r, d), lambda i: (i, 0)),
            pl.BlockSpec((d, d), lambda i: (0, 0)),
            pl.BlockSpec((1, d), lambda i: (0, 0)),
            pl.BlockSpec((d, d), lambda i: (0, 0)),
            pl.BlockSpec((1, d), lambda i: (0, 0)),
        ],
        out_specs=pl.BlockSpec((r, d), lambda i: (i, 0)),
        out_shape=jax.ShapeDtypeStruct((n, d), jnp.float32),
    )(x, p0, p1, W1, b1.reshape(1, d), W2, b2.reshape(1, d))


def kernel(x, edge_index, W1, b1, W2, b2):
    n, d = x.shape
    e = edge_index.shape[1]

    rps = -(-(n + 1) // _NS)
    rps = -(-rps // 8) * 8
    n_pad = _NS * rps

    eb = _NC * _NS * _K
    e_pad = -(-e // eb) * eb
    src = edge_index[0]
    dst = edge_index[1]
    if e_pad != e:
        pad = e_pad - e
        src = jnp.concatenate([src, jnp.zeros((pad,), jnp.int32)])
        dst = jnp.concatenate([dst, jnp.full((pad,), n, jnp.int32)])
    bpw = e_pad // (_NC * _NS * _K)
    src = src.reshape(e_pad // _K, 1, _K)
    dst = dst.reshape(e_pad // _K, 1, _K)

    zeros = jnp.zeros((rps, d), jnp.float32)
    partials = _sc_aggregate(x, src, dst, zeros, n_pad=n_pad, rps=rps, bpw=bpw)
    p0 = partials[:n]
    p1 = partials[n_pad:n_pad + n]
    return _mlp(x, p0, p1, W1, b1, W2, b2)
